# Initial kernel scaffold; baseline (speedup 1.0000x reference)
import jax
import jax.numpy as jnp
from jax.experimental import pallas as pl


def kernel(atom_features, edge_index, pair_indices, pair_features, W_emb, b_emb, conv_W, conv_b, bn_gamma, bn_beta, mlp_W1, mlp_b1, mlp_W2, mlp_b2, mlp_W3, mlp_b3):
    raise NotImplementedError("write your pallas kernel here")


# speedup vs baseline: 9.6811x; 9.6811x over previous
import functools

import jax
import jax.numpy as jnp
from jax import lax
from jax.experimental import pallas as pl
from jax.experimental.pallas import tpu as pltpu
from jax.experimental.pallas import tpu_sc as plsc

N = 10000
E = 320000
P = 50000
H = 128
DP = 16
L = 3

NC = 2
NS = 16
NW = NC * NS

EB = 80
ET = E // NW
NP = 10240
ROWS_T = NP // NS
RB = 80
RCH = ROWS_T // RB

PADP = 100352
PT = PADP // NW
PB = 112

_F32 = jnp.float32


def _mesh():
    return plsc.VectorSubcoreMesh(
        core_axis_name="c", subcore_axis_name="s",
        num_cores=NC, num_subcores=NS)


def _sc_deg(dst, zeros16, ones16):
    @functools.partial(
        pl.kernel,
        out_type=jax.ShapeDtypeStruct((NC, NP, 16), _F32),
        mesh=_mesh(),
        scratch_types=[
            pltpu.VMEM((EB,), jnp.int32),
            pltpu.VMEM((EB, 16), _F32),
            pltpu.VMEM((RB, 16), _F32),
            pltpu.VMEM_SHARED((NP, 16), _F32),
        ],
    )
    def k(dst_hbm, z_hbm, one_hbm, out_hbm, dst_v, ones_v, stage_v, acc):
        c = lax.axis_index("c")
        s = lax.axis_index("s")
        wid = s * NC + c
        r0 = s * ROWS_T
        pltpu.sync_copy(one_hbm, ones_v)
        for j in range(RCH):
            pltpu.sync_copy(z_hbm.at[pl.ds(r0 + j * RB, RB)], stage_v)
            pltpu.sync_copy(stage_v, acc.at[pl.ds(r0 + j * RB, RB)])
        plsc.subcore_barrier()

        def body(i, carry):
            off = wid * ET + i * EB
            pltpu.sync_copy(dst_hbm.at[pl.ds(off, EB)], dst_v)
            pltpu.sync_copy(ones_v, acc.at[dst_v], add=True)
            return carry

        lax.fori_loop(0, ET // EB, body, 0)
        plsc.subcore_barrier()
        for j in range(RCH):
            pltpu.sync_copy(acc.at[pl.ds(r0 + j * RB, RB)], stage_v)
            pltpu.sync_copy(stage_v, out_hbm.at[c, pl.ds(r0 + j * RB, RB)])

    return k(dst, zeros16, ones16)


def _sc_msg(src, dst, hp, zeros128):
    @functools.partial(
        pl.kernel,
        out_type=jax.ShapeDtypeStruct((NC, NP, H), _F32),
        mesh=_mesh(),
        scratch_types=[
            pltpu.VMEM((EB,), jnp.int32),
            pltpu.VMEM((EB,), jnp.int32),
            pltpu.VMEM((EB, H), _F32),
            pltpu.VMEM((RB, H), _F32),
            pltpu.VMEM_SHARED((NP, H), _F32),
            pltpu.SemaphoreType.DMA,
        ],
    )
    def k(src_hbm, dst_hbm, h_hbm, z_hbm, out_hbm,
          src_v, dst_v, rows_v, stage_v, acc, sem):
        c = lax.axis_index("c")
        s = lax.axis_index("s")
        wid = s * NC + c
        r0 = s * ROWS_T
        for j in range(RCH):
            pltpu.sync_copy(z_hbm.at[pl.ds(r0 + j * RB, RB)], stage_v)
            pltpu.sync_copy(stage_v, acc.at[pl.ds(r0 + j * RB, RB)])
        plsc.subcore_barrier()

        def body(i, carry):
            off = wid * ET + i * EB
            pltpu.sync_copy(src_hbm.at[pl.ds(off, EB)], src_v)
            pltpu.sync_copy(dst_hbm.at[pl.ds(off, EB)], dst_v)
            pltpu.async_copy(h_hbm.at[src_v], rows_v, sem).wait()
            pltpu.sync_copy(rows_v, acc.at[dst_v], add=True)
            return carry

        lax.fori_loop(0, ET // EB, body, 0)
        plsc.subcore_barrier()
        for j in range(RCH):
            pltpu.sync_copy(acc.at[pl.ds(r0 + j * RB, RB)], stage_v)
            pltpu.sync_copy(stage_v, out_hbm.at[c, pl.ds(r0 + j * RB, RB)])

    return k(src, dst, hp, zeros128)


def _sc_pairs(table, cidx):
    @functools.partial(
        pl.kernel,
        out_type=jax.ShapeDtypeStruct((PADP, H), _F32),
        mesh=_mesh(),
        scratch_types=[
            pltpu.VMEM((PB,), jnp.int32),
            pltpu.VMEM((PB, H), _F32),
            pltpu.SemaphoreType.DMA,
        ],
    )
    def k(tab_hbm, idx_hbm, out_hbm, idx_v, rows_v, sem):
        c = lax.axis_index("c")
        s = lax.axis_index("s")
        wid = s * NC + c

        def body(i, carry):
            off = wid * PT + i * PB
            pltpu.sync_copy(idx_hbm.at[pl.ds(off, PB)], idx_v)
            pltpu.async_copy(tab_hbm.at[idx_v], rows_v, sem).wait()
            pltpu.sync_copy(rows_v, out_hbm.at[pl.ds(off, PB)])
            return carry

        lax.fori_loop(0, PT // PB, body, 0)

    return k(table, cidx)


def _tc_pre(atom, W_emb, b_emb, degp, W0):
    def body(a_ref, we_ref, be_ref, dp_ref, w0_ref, hp_ref, dis_ref):
        deg = dp_ref[0] + dp_ref[1] + 1.0
        dis = lax.rsqrt(deg)
        dis_ref[...] = dis
        x0 = jnp.dot(a_ref[...], we_ref[...],
                     preferred_element_type=_F32) + be_ref[...]
        hp_ref[...] = jnp.dot(x0, w0_ref[...],
                              preferred_element_type=_F32) * dis[:, :1]

    return pl.pallas_call(
        body,
        grid=(1,),
        in_specs=[
            pl.BlockSpec((N, H), lambda i: (0, 0)),
            pl.BlockSpec((H, H), lambda i: (0, 0)),
            pl.BlockSpec((1, H), lambda i: (0, 0)),
            pl.BlockSpec((NC, N, 16), lambda i: (0, 0, 0)),
            pl.BlockSpec((H, H), lambda i: (0, 0)),
        ],
        out_specs=(pl.BlockSpec((N, H), lambda i: (0, 0)),
                   pl.BlockSpec((N, 16), lambda i: (0, 0))),
        out_shape=(jax.ShapeDtypeStruct((N, H), _F32),
                   jax.ShapeDtypeStruct((N, 16), _F32)),
    )(atom, W_emb, b_emb, degp, W0)


def _bn_relu(p0, p1, hp, d, b, g, bt):
    out = (p0 + p1 + hp) * d + b
    m = jnp.mean(out, axis=0, keepdims=True)
    cen = out - m
    v = jnp.mean(cen * cen, axis=0, keepdims=True)
    xn = cen * lax.rsqrt(v + 1e-5) * g + bt
    return jnp.maximum(xn, 0.0)


def _tc_mid(p, hp, dis, b, g, bt, W_next):
    def body(p_ref, hp_ref, dis_ref, b_ref, g_ref, bt_ref, w_ref, o_ref):
        d = dis_ref[...][:, :1]
        x = _bn_relu(p_ref[0], p_ref[1], hp_ref[...], d,
                     b_ref[...], g_ref[...], bt_ref[...])
        o_ref[...] = jnp.dot(x, w_ref[...], preferred_element_type=_F32) * d

    return pl.pallas_call(
        body,
        grid=(1,),
        in_specs=[
            pl.BlockSpec((NC, N, H), lambda i: (0, 0, 0)),
            pl.BlockSpec((N, H), lambda i: (0, 0)),
            pl.BlockSpec((N, 16), lambda i: (0, 0)),
            pl.BlockSpec((1, H), lambda i: (0, 0)),
            pl.BlockSpec((1, H), lambda i: (0, 0)),
            pl.BlockSpec((1, H), lambda i: (0, 0)),
            pl.BlockSpec((H, H), lambda i: (0, 0)),
        ],
        out_specs=pl.BlockSpec((N, H), lambda i: (0, 0)),
        out_shape=jax.ShapeDtypeStruct((N, H), _F32),
    )(p, hp, dis, b, g, bt, W_next)


def _tc_last(p, hp, dis, b, g, bt, W1a, W1b):
    def body(p_ref, hp_ref, dis_ref, b_ref, g_ref, bt_ref,
             wa_ref, wb_ref, t_ref):
        d = dis_ref[...][:, :1]
        x = _bn_relu(p_ref[0], p_ref[1], hp_ref[...], d,
                     b_ref[...], g_ref[...], bt_ref[...])
        t_ref[0:N, :] = jnp.dot(x, wa_ref[...], preferred_element_type=_F32)
        t_ref[N:2 * N, :] = jnp.dot(x, wb_ref[...],
                                    preferred_element_type=_F32)

    return pl.pallas_call(
        body,
        grid=(1,),
        in_specs=[
            pl.BlockSpec((NC, N, H), lambda i: (0, 0, 0)),
            pl.BlockSpec((N, H), lambda i: (0, 0)),
            pl.BlockSpec((N, 16), lambda i: (0, 0)),
            pl.BlockSpec((1, H), lambda i: (0, 0)),
            pl.BlockSpec((1, H), lambda i: (0, 0)),
            pl.BlockSpec((1, H), lambda i: (0, 0)),
            pl.BlockSpec((H, H), lambda i: (0, 0)),
            pl.BlockSpec((H, H), lambda i: (0, 0)),
        ],
        out_specs=pl.BlockSpec((2 * N, H), lambda i: (0, 0)),
        out_shape=jax.ShapeDtypeStruct((2 * N, H), _F32),
    )(p, hp, dis, b, g, bt, W1a, W1b)


def _tc_mlp(R, pf, W1c, b1, W2, b2, W3, b3):
    BM = 2000
    nblk = P // BM

    def body(a_ref, b_ref, pf_ref, w1c_ref, b1_ref, w2_ref, b2_ref,
             w3_ref, b3_ref, o_ref):
        h1 = jnp.maximum(
            a_ref[...] + b_ref[...]
            + jnp.dot(pf_ref[...], w1c_ref[...], preferred_element_type=_F32)
            + b1_ref[...], 0.0)
        h2 = jnp.maximum(
            jnp.dot(h1, w2_ref[...], preferred_element_type=_F32)
            + b2_ref[...], 0.0)
        o_ref[...] = jnp.dot(h2, w3_ref[...],
                             preferred_element_type=_F32) + b3_ref[...]

    return pl.pallas_call(
        body,
        grid=(nblk,),
        in_specs=[
            pl.BlockSpec((BM, H), lambda i: (i, 0)),
            pl.BlockSpec((BM, H), lambda i: (i + nblk, 0)),
            pl.BlockSpec((BM, DP), lambda i: (i, 0)),
            pl.BlockSpec((DP, H), lambda i: (0, 0)),
            pl.BlockSpec((1, H), lambda i: (0, 0)),
            pl.BlockSpec((H, H // 2), lambda i: (0, 0)),
            pl.BlockSpec((1, H // 2), lambda i: (0, 0)),
            pl.BlockSpec((H // 2, 1), lambda i: (0, 0)),
            pl.BlockSpec((1, 1), lambda i: (0, 0)),
        ],
        out_specs=pl.BlockSpec((BM, 1), lambda i: (i, 0)),
        out_shape=jax.ShapeDtypeStruct((P, 1), _F32),
    )(R, R, pf, W1c, b1, W2, b2, W3, b3)


def kernel(atom_features, edge_index, pair_indices, pair_features,
           W_emb, b_emb, conv_W, conv_b, bn_gamma, bn_beta,
           mlp_W1, mlp_b1, mlp_W2, mlp_b2, mlp_W3, mlp_b3):
    src = edge_index[0]
    dst = edge_index[1]

    zeros16 = jnp.zeros((NP, 16), _F32)
    ones16 = jnp.ones((EB, 16), _F32)
    zeros128 = jnp.zeros((NP, H), _F32)

    degp = _sc_deg(dst, zeros16, ones16)
    hp, dis = _tc_pre(atom_features, W_emb, b_emb[None, :], degp, conv_W[0])

    W1a = mlp_W1[0:H]
    W1b = mlp_W1[H:2 * H]
    W1c = mlp_W1[2 * H:]

    for i in range(L):
        pmsg = _sc_msg(src, dst, hp, zeros128)
        if i < L - 1:
            hp = _tc_mid(pmsg, hp, dis, conv_b[i][None, :],
                         bn_gamma[i][None, :], bn_beta[i][None, :],
                         conv_W[i + 1])
        else:
            T = _tc_last(pmsg, hp, dis, conv_b[i][None, :],
                         bn_gamma[i][None, :], bn_beta[i][None, :],
                         W1a, W1b)

    cidx = jnp.concatenate([
        pair_indices[:, 0],
        pair_indices[:, 1] + N,
        jnp.zeros((PADP - 2 * P,), jnp.int32),
    ])
    R = _sc_pairs(T, cidx)

    return _tc_mlp(R, pair_features, W1c, mlp_b1[None, :],
                   mlp_W2, mlp_b2[None, :], mlp_W3, mlp_b3[None, :])
